# baseline (speedup 1.0000x reference)
import functools

import jax
import jax.numpy as jnp
from jax import lax
from jax.experimental import pallas as pl
from jax.experimental.pallas import tpu as pltpu
from jax.experimental.pallas import tpu_sc as plsc

MAX_SEQS = 1024
MAX_TOKENS = 8192
NUM_NEW = 1024

_GROUPS = NUM_NEW // 16
_BLK_ROWS = 64
_NBLK = MAX_SEQS // _BLK_ROWS
_BKT_SHIFT = 6


def _sc_plan_body(num_tokens_hbm, sids_hbm, ntok_hbm, nlp_hbm,
                  cnt_out, starts_out, r8_out, cb_out, sel_out, tok_out,
                  lp_out,
                  sids_v, ntok_v, nlp_v, cnt_v, pos_v, bkt_v,
                  sr8_v, scb_v, ssel_v, stok_v, slp_v, st_v):
  @pl.when((lax.axis_index("c") == 0) & (lax.axis_index("s") == 0))
  def _():
    pltpu.sync_copy(sids_hbm, sids_v)
    pltpu.sync_copy(num_tokens_hbm, cnt_v)
    pltpu.sync_copy(ntok_hbm, ntok_v)
    pltpu.sync_copy(nlp_hbm, nlp_v)

    bkt_v[...] = jnp.zeros((16,), jnp.int32)

    @pl.loop(0, _GROUPS)
    def _(g):
      v = sids_v[pl.ds(g * 16, 16)]
      r, is_last = plsc.scan_count(v)
      base = plsc.load_gather(cnt_v, [v])
      pos = base + r - 1
      plsc.store_scatter(cnt_v, [v], pos + 1, mask=is_last)
      pos_v[pl.ds(g * 16, 16)] = pos

      b = lax.shift_right_logical(v, _BKT_SHIFT)
      rb, lastb = plsc.scan_count(b)
      bbase = plsc.load_gather(bkt_v, [b])
      plsc.store_scatter(bkt_v, [b], bbase + rb, mask=lastb)

    counts = bkt_v[...]
    incl = plsc.cumsum(counts)
    excl = incl - counts
    st_v[pl.ds(0, 16)] = excl
    st_v[pl.ds(16, 16)] = jnp.full((16,), NUM_NEW, jnp.int32)
    bkt_v[...] = excl

    @pl.loop(0, _GROUPS)
    def _(g):
      v = sids_v[pl.ds(g * 16, 16)]
      b = lax.shift_right_logical(v, _BKT_SHIFT)
      rb, lastb = plsc.scan_count(b)
      dbase = plsc.load_gather(bkt_v, [b])
      dest = dbase + rb - 1
      plsc.store_scatter(bkt_v, [b], dbase + rb, mask=lastb)

      row = jnp.bitwise_and(v, _BLK_ROWS - 1)
      pos = pos_v[pl.ds(g * 16, 16)]
      r8 = jnp.bitwise_and(row, ~7)
      cb = jnp.bitwise_and(pos, ~127)
      selid = jnp.bitwise_and(row, 7) * 128 + (pos - cb)
      plsc.store_scatter(sr8_v, [dest], r8)
      plsc.store_scatter(scb_v, [dest], cb)
      plsc.store_scatter(ssel_v, [dest], selid)
      plsc.store_scatter(stok_v, [dest], ntok_v[pl.ds(g * 16, 16)])
      plsc.store_scatter(slp_v, [dest], nlp_v[pl.ds(g * 16, 16)])

    pltpu.sync_copy(cnt_v, cnt_out)
    pltpu.sync_copy(st_v, starts_out)
    pltpu.sync_copy(sr8_v, r8_out)
    pltpu.sync_copy(scb_v, cb_out)
    pltpu.sync_copy(ssel_v, sel_out)
    pltpu.sync_copy(stok_v, tok_out)
    pltpu.sync_copy(slp_v, lp_out)


def _sc_plan(num_tokens, sids, ntok, nlp):
  mesh = plsc.VectorSubcoreMesh(core_axis_name="c", subcore_axis_name="s")
  return pl.kernel(
      _sc_plan_body,
      out_type=(
          jax.ShapeDtypeStruct((MAX_SEQS,), jnp.int32),
          jax.ShapeDtypeStruct((32,), jnp.int32),
          jax.ShapeDtypeStruct((NUM_NEW,), jnp.int32),
          jax.ShapeDtypeStruct((NUM_NEW,), jnp.int32),
          jax.ShapeDtypeStruct((NUM_NEW,), jnp.int32),
          jax.ShapeDtypeStruct((NUM_NEW,), jnp.int32),
          jax.ShapeDtypeStruct((NUM_NEW,), jnp.float32),
      ),
      mesh=mesh,
      compiler_params=pltpu.CompilerParams(needs_layout_passes=False),
      scratch_types=[
          pltpu.VMEM((NUM_NEW,), jnp.int32),
          pltpu.VMEM((NUM_NEW,), jnp.int32),
          pltpu.VMEM((NUM_NEW,), jnp.float32),
          pltpu.VMEM((MAX_SEQS,), jnp.int32),
          pltpu.VMEM((NUM_NEW,), jnp.int32),
          pltpu.VMEM((16,), jnp.int32),
          pltpu.VMEM((NUM_NEW,), jnp.int32),
          pltpu.VMEM((NUM_NEW,), jnp.int32),
          pltpu.VMEM((NUM_NEW,), jnp.int32),
          pltpu.VMEM((NUM_NEW,), jnp.int32),
          pltpu.VMEM((NUM_NEW,), jnp.float32),
          pltpu.VMEM((32,), jnp.int32),
      ],
  )(num_tokens, sids, ntok, nlp)


def _tc_apply_body(starts_ref, r8_ref, cb_ref, sel_ref, tokv_ref, lpv_ref,
                   tok_in, lp_in, tok_out, lp_out):
  b = pl.program_id(0)
  tok_out[...] = tok_in[...]
  lp_out[...] = lp_in[...]

  pid = (lax.broadcasted_iota(jnp.int32, (8, 128), 0) * 128
         + lax.broadcasted_iota(jnp.int32, (8, 128), 1))

  def body(j, _):
    r8 = pl.multiple_of(r8_ref[j], 8)
    cb = pl.multiple_of(cb_ref[j], 128)
    sel = pid == sel_ref[j]

    cur_t = tok_out[pl.ds(r8, 8), pl.ds(cb, 128)]
    tok_out[pl.ds(r8, 8), pl.ds(cb, 128)] = jnp.where(sel, tokv_ref[j], cur_t)
    cur_l = lp_out[pl.ds(r8, 8), pl.ds(cb, 128)]
    lp_out[pl.ds(r8, 8), pl.ds(cb, 128)] = jnp.where(sel, lpv_ref[j], cur_l)
    return 0

  lax.fori_loop(starts_ref[b], starts_ref[b + 1], body, 0)


def _tc_apply(starts, r8, cb, selid, tokv, lpv, tokens, logprobs):
  smem = pl.BlockSpec(memory_space=pltpu.SMEM)
  blk = pl.BlockSpec((_BLK_ROWS, MAX_TOKENS), lambda i: (i, 0))
  return pl.pallas_call(
      _tc_apply_body,
      grid=(_NBLK,),
      in_specs=[smem, smem, smem, smem, smem, smem, blk, blk],
      out_specs=[blk, blk],
      out_shape=(
          jax.ShapeDtypeStruct((MAX_SEQS, MAX_TOKENS), jnp.int32),
          jax.ShapeDtypeStruct((MAX_SEQS, MAX_TOKENS), jnp.float32),
      ),
      compiler_params=pltpu.CompilerParams(
          dimension_semantics=("arbitrary",),
      ),
  )(starts, r8, cb, selid, tokv, lpv, tokens, logprobs)


@jax.jit
def _kernel_impl(tokens, logprobs, num_tokens, local_seq_ids, new_tokens,
                 new_log_probs):
  cnt, starts, r8, cb, selid, tokv, lpv = _sc_plan(
      num_tokens, local_seq_ids, new_tokens, new_log_probs)
  tok_out, lp_out = _tc_apply(starts, r8, cb, selid, tokv, lpv, tokens,
                              logprobs)
  return tok_out, lp_out, cnt


def kernel(tokens, logprobs, num_tokens, local_seq_ids, new_tokens,
           new_log_probs, num_new_tokens):
  del num_new_tokens
  return _kernel_impl(tokens, logprobs, num_tokens, local_seq_ids, new_tokens,
                      new_log_probs)


# speedup vs baseline: 1.0485x; 1.0485x over previous
import functools

import jax
import jax.numpy as jnp
from jax import lax
from jax.experimental import pallas as pl
from jax.experimental.pallas import tpu as pltpu
from jax.experimental.pallas import tpu_sc as plsc

MAX_SEQS = 1024
MAX_TOKENS = 8192
NUM_NEW = 1024

_GROUPS = NUM_NEW // 16
_BLK_ROWS = 128
_NBLK = MAX_SEQS // _BLK_ROWS


def _sc_plan_body(num_tokens_hbm, sids_hbm, ntok_hbm, nlp_hbm,
                  cnt_out, starts_out, r8_out, cb_out, sel_out, tok_out,
                  lp_out,
                  sids_v, ntok_v, nlp_v, cnt_v, pos_v, bkt_v,
                  sr8_v, scb_v, ssel_v, stok_v, slp_v, st_v,
                  sem_a, sem_b, sem_o):
  @pl.when((lax.axis_index("c") == 0) & (lax.axis_index("s") == 0))
  def _():
    in_a = [pltpu.async_copy(sids_hbm, sids_v, sem_a),
            pltpu.async_copy(num_tokens_hbm, cnt_v, sem_a)]
    in_b = [pltpu.async_copy(ntok_hbm, ntok_v, sem_b),
            pltpu.async_copy(nlp_hbm, nlp_v, sem_b)]

    bkt_v[...] = jnp.zeros((16,), jnp.int32)
    for cp in in_a:
      cp.wait()

    @pl.loop(0, _GROUPS)
    def _(g):
      v = sids_v[pl.ds(g * 16, 16)]
      r, is_last = plsc.scan_count(v)
      base = plsc.load_gather(cnt_v, [v])
      pos = base + r - 1
      plsc.store_scatter(cnt_v, [v], pos + 1, mask=is_last)
      pos_v[pl.ds(g * 16, 16)] = pos

      b = lax.shift_right_logical(v, 7)
      rb, lastb = plsc.scan_count(b)
      bbase = plsc.load_gather(bkt_v, [b])
      plsc.store_scatter(bkt_v, [b], bbase + rb, mask=lastb)

    out_cnt = pltpu.async_copy(cnt_v, cnt_out, sem_o)

    counts = bkt_v[...]
    incl = plsc.cumsum(counts)
    excl = incl - counts
    lane = lax.iota(jnp.int32, 16)
    st_v[...] = jnp.where(lane >= _NBLK, NUM_NEW, excl)
    bkt_v[...] = excl
    out_st = pltpu.async_copy(st_v, starts_out, sem_o)
    for cp in in_b:
      cp.wait()

    @pl.loop(0, _GROUPS)
    def _(g):
      v = sids_v[pl.ds(g * 16, 16)]
      b = lax.shift_right_logical(v, 7)
      rb, lastb = plsc.scan_count(b)
      dbase = plsc.load_gather(bkt_v, [b])
      dest = dbase + rb - 1
      plsc.store_scatter(bkt_v, [b], dbase + rb, mask=lastb)

      row = jnp.bitwise_and(v, _BLK_ROWS - 1)
      pos = pos_v[pl.ds(g * 16, 16)]
      r8 = jnp.bitwise_and(row, ~7)
      cb = jnp.bitwise_and(pos, ~127)
      selid = jnp.bitwise_and(row, 7) * 128 + (pos - cb)
      plsc.store_scatter(sr8_v, [dest], r8)
      plsc.store_scatter(scb_v, [dest], cb)
      plsc.store_scatter(ssel_v, [dest], selid)
      plsc.store_scatter(stok_v, [dest], ntok_v[pl.ds(g * 16, 16)])
      plsc.store_scatter(slp_v, [dest], nlp_v[pl.ds(g * 16, 16)])

    outs = [pltpu.async_copy(sr8_v, r8_out, sem_o),
            pltpu.async_copy(scb_v, cb_out, sem_o),
            pltpu.async_copy(ssel_v, sel_out, sem_o),
            pltpu.async_copy(stok_v, tok_out, sem_o),
            pltpu.async_copy(slp_v, lp_out, sem_o)]
    out_cnt.wait()
    out_st.wait()
    for cp in outs:
      cp.wait()


def _sc_plan(num_tokens, sids, ntok, nlp):
  mesh = plsc.VectorSubcoreMesh(core_axis_name="c", subcore_axis_name="s")
  return pl.kernel(
      _sc_plan_body,
      out_type=(
          jax.ShapeDtypeStruct((MAX_SEQS,), jnp.int32),
          jax.ShapeDtypeStruct((16,), jnp.int32),
          jax.ShapeDtypeStruct((NUM_NEW,), jnp.int32),
          jax.ShapeDtypeStruct((NUM_NEW,), jnp.int32),
          jax.ShapeDtypeStruct((NUM_NEW,), jnp.int32),
          jax.ShapeDtypeStruct((NUM_NEW,), jnp.int32),
          jax.ShapeDtypeStruct((NUM_NEW,), jnp.float32),
      ),
      mesh=mesh,
      compiler_params=pltpu.CompilerParams(needs_layout_passes=False),
      scratch_types=[
          pltpu.VMEM((NUM_NEW,), jnp.int32),
          pltpu.VMEM((NUM_NEW,), jnp.int32),
          pltpu.VMEM((NUM_NEW,), jnp.float32),
          pltpu.VMEM((MAX_SEQS,), jnp.int32),
          pltpu.VMEM((NUM_NEW,), jnp.int32),
          pltpu.VMEM((16,), jnp.int32),
          pltpu.VMEM((NUM_NEW,), jnp.int32),
          pltpu.VMEM((NUM_NEW,), jnp.int32),
          pltpu.VMEM((NUM_NEW,), jnp.int32),
          pltpu.VMEM((NUM_NEW,), jnp.int32),
          pltpu.VMEM((NUM_NEW,), jnp.float32),
          pltpu.VMEM((16,), jnp.int32),
          pltpu.SemaphoreType.DMA,
          pltpu.SemaphoreType.DMA,
          pltpu.SemaphoreType.DMA,
      ],
  )(num_tokens, sids, ntok, nlp)


def _tc_apply_body(starts_ref, r8_ref, cb_ref, sel_ref, tokv_ref, lpv_ref,
                   tok_in, lp_in, tok_out, lp_out):
  b = pl.program_id(0)
  tok_out[...] = tok_in[...]
  lp_out[...] = lp_in[...]

  pid = (lax.broadcasted_iota(jnp.int32, (8, 128), 0) * 128
         + lax.broadcasted_iota(jnp.int32, (8, 128), 1))

  def body(j, _):
    r8 = pl.multiple_of(r8_ref[j], 8)
    cb = pl.multiple_of(cb_ref[j], 128)
    sel = pid == sel_ref[j]

    cur_t = tok_out[pl.ds(r8, 8), pl.ds(cb, 128)]
    tok_out[pl.ds(r8, 8), pl.ds(cb, 128)] = jnp.where(sel, tokv_ref[j], cur_t)
    cur_l = lp_out[pl.ds(r8, 8), pl.ds(cb, 128)]
    lp_out[pl.ds(r8, 8), pl.ds(cb, 128)] = jnp.where(sel, lpv_ref[j], cur_l)
    return 0

  lax.fori_loop(starts_ref[b], starts_ref[b + 1], body, 0)


def _tc_apply(starts, r8, cb, selid, tokv, lpv, tokens, logprobs):
  smem = pl.BlockSpec(memory_space=pltpu.SMEM)
  blk = pl.BlockSpec((_BLK_ROWS, MAX_TOKENS), lambda i: (i, 0))
  return pl.pallas_call(
      _tc_apply_body,
      grid=(_NBLK,),
      in_specs=[smem, smem, smem, smem, smem, smem, blk, blk],
      out_specs=[blk, blk],
      out_shape=(
          jax.ShapeDtypeStruct((MAX_SEQS, MAX_TOKENS), jnp.int32),
          jax.ShapeDtypeStruct((MAX_SEQS, MAX_TOKENS), jnp.float32),
      ),
      compiler_params=pltpu.CompilerParams(
          dimension_semantics=("arbitrary",),
      ),
  )(starts, r8, cb, selid, tokv, lpv, tokens, logprobs)


@jax.jit
def _kernel_impl(tokens, logprobs, num_tokens, local_seq_ids, new_tokens,
                 new_log_probs):
  cnt, starts, r8, cb, selid, tokv, lpv = _sc_plan(
      num_tokens, local_seq_ids, new_tokens, new_log_probs)
  tok_out, lp_out = _tc_apply(starts, r8, cb, selid, tokv, lpv, tokens,
                              logprobs)
  return tok_out, lp_out, cnt


def kernel(tokens, logprobs, num_tokens, local_seq_ids, new_tokens,
           new_log_probs, num_new_tokens):
  del num_new_tokens
  return _kernel_impl(tokens, logprobs, num_tokens, local_seq_ids, new_tokens,
                      new_log_probs)


# speedup vs baseline: 1.0521x; 1.0034x over previous
import functools

import jax
import jax.numpy as jnp
from jax import lax
from jax.experimental import pallas as pl
from jax.experimental.pallas import tpu as pltpu
from jax.experimental.pallas import tpu_sc as plsc

MAX_SEQS = 1024
MAX_TOKENS = 8192
NUM_NEW = 1024

_GROUPS = NUM_NEW // 16
_BLK_ROWS = 128
_NBLK = MAX_SEQS // _BLK_ROWS


def _sc_plan_body(num_tokens_hbm, sids_hbm, ntok_hbm, nlp_hbm,
                  cnt_out, starts_out, r8_out, cb_out, sel_out, tok_out,
                  lp_out,
                  sids_v, ntok_v, nlp_v, cnt_v, pos_v, bkt_v,
                  sr8_v, scb_v, ssel_v, stok_v, slp_v, st_v,
                  sem_a, sem_b, sem_o):
  @pl.when((lax.axis_index("c") == 0) & (lax.axis_index("s") == 0))
  def _():
    in_a = [pltpu.async_copy(sids_hbm, sids_v, sem_a),
            pltpu.async_copy(num_tokens_hbm, cnt_v, sem_a)]
    in_b = [pltpu.async_copy(ntok_hbm, ntok_v, sem_b),
            pltpu.async_copy(nlp_hbm, nlp_v, sem_b)]

    bkt_v[...] = jnp.zeros((16,), jnp.int32)
    for cp in in_a:
      cp.wait()

    @pl.loop(0, _GROUPS, unroll=4)
    def _(g):
      v = sids_v[pl.ds(g * 16, 16)]
      r, is_last = plsc.scan_count(v)
      base = plsc.load_gather(cnt_v, [v])
      pos = base + r - 1
      plsc.store_scatter(cnt_v, [v], pos + 1, mask=is_last)
      pos_v[pl.ds(g * 16, 16)] = pos

      b = lax.shift_right_logical(v, 7)
      rb, lastb = plsc.scan_count(b)
      bbase = plsc.load_gather(bkt_v, [b])
      plsc.store_scatter(bkt_v, [b], bbase + rb, mask=lastb)

    out_cnt = pltpu.async_copy(cnt_v, cnt_out, sem_o)

    counts = bkt_v[...]
    incl = plsc.cumsum(counts)
    excl = incl - counts
    lane = lax.iota(jnp.int32, 16)
    st_v[...] = jnp.where(lane >= _NBLK, NUM_NEW, excl)
    bkt_v[...] = excl
    out_st = pltpu.async_copy(st_v, starts_out, sem_o)
    for cp in in_b:
      cp.wait()

    @pl.loop(0, _GROUPS, unroll=4)
    def _(g):
      v = sids_v[pl.ds(g * 16, 16)]
      b = lax.shift_right_logical(v, 7)
      rb, lastb = plsc.scan_count(b)
      dbase = plsc.load_gather(bkt_v, [b])
      dest = dbase + rb - 1
      plsc.store_scatter(bkt_v, [b], dbase + rb, mask=lastb)

      row = jnp.bitwise_and(v, _BLK_ROWS - 1)
      pos = pos_v[pl.ds(g * 16, 16)]
      r8 = jnp.bitwise_and(row, ~7)
      cb = jnp.bitwise_and(pos, ~127)
      selid = jnp.bitwise_and(row, 7) * 128 + (pos - cb)
      plsc.store_scatter(sr8_v, [dest], r8)
      plsc.store_scatter(scb_v, [dest], cb)
      plsc.store_scatter(ssel_v, [dest], selid)
      plsc.store_scatter(stok_v, [dest], ntok_v[pl.ds(g * 16, 16)])
      plsc.store_scatter(slp_v, [dest], nlp_v[pl.ds(g * 16, 16)])

    outs = [pltpu.async_copy(sr8_v, r8_out, sem_o),
            pltpu.async_copy(scb_v, cb_out, sem_o),
            pltpu.async_copy(ssel_v, sel_out, sem_o),
            pltpu.async_copy(stok_v, tok_out, sem_o),
            pltpu.async_copy(slp_v, lp_out, sem_o)]
    out_cnt.wait()
    out_st.wait()
    for cp in outs:
      cp.wait()


def _sc_plan(num_tokens, sids, ntok, nlp):
  mesh = plsc.VectorSubcoreMesh(core_axis_name="c", subcore_axis_name="s")
  return pl.kernel(
      _sc_plan_body,
      out_type=(
          jax.ShapeDtypeStruct((MAX_SEQS,), jnp.int32),
          jax.ShapeDtypeStruct((16,), jnp.int32),
          jax.ShapeDtypeStruct((NUM_NEW,), jnp.int32),
          jax.ShapeDtypeStruct((NUM_NEW,), jnp.int32),
          jax.ShapeDtypeStruct((NUM_NEW,), jnp.int32),
          jax.ShapeDtypeStruct((NUM_NEW,), jnp.int32),
          jax.ShapeDtypeStruct((NUM_NEW,), jnp.float32),
      ),
      mesh=mesh,
      compiler_params=pltpu.CompilerParams(needs_layout_passes=False),
      scratch_types=[
          pltpu.VMEM((NUM_NEW,), jnp.int32),
          pltpu.VMEM((NUM_NEW,), jnp.int32),
          pltpu.VMEM((NUM_NEW,), jnp.float32),
          pltpu.VMEM((MAX_SEQS,), jnp.int32),
          pltpu.VMEM((NUM_NEW,), jnp.int32),
          pltpu.VMEM((16,), jnp.int32),
          pltpu.VMEM((NUM_NEW,), jnp.int32),
          pltpu.VMEM((NUM_NEW,), jnp.int32),
          pltpu.VMEM((NUM_NEW,), jnp.int32),
          pltpu.VMEM((NUM_NEW,), jnp.int32),
          pltpu.VMEM((NUM_NEW,), jnp.float32),
          pltpu.VMEM((16,), jnp.int32),
          pltpu.SemaphoreType.DMA,
          pltpu.SemaphoreType.DMA,
          pltpu.SemaphoreType.DMA,
      ],
  )(num_tokens, sids, ntok, nlp)


def _tc_apply_body(starts_ref, r8_ref, cb_ref, sel_ref, tokv_ref, lpv_ref,
                   tok_in, lp_in, tok_out, lp_out):
  b = pl.program_id(0)
  tok_out[...] = tok_in[...]
  lp_out[...] = lp_in[...]

  pid = (lax.broadcasted_iota(jnp.int32, (8, 128), 0) * 128
         + lax.broadcasted_iota(jnp.int32, (8, 128), 1))

  def body(j, _):
    r8 = pl.multiple_of(r8_ref[j], 8)
    cb = pl.multiple_of(cb_ref[j], 128)
    sel = pid == sel_ref[j]

    cur_t = tok_out[pl.ds(r8, 8), pl.ds(cb, 128)]
    tok_out[pl.ds(r8, 8), pl.ds(cb, 128)] = jnp.where(sel, tokv_ref[j], cur_t)
    cur_l = lp_out[pl.ds(r8, 8), pl.ds(cb, 128)]
    lp_out[pl.ds(r8, 8), pl.ds(cb, 128)] = jnp.where(sel, lpv_ref[j], cur_l)
    return 0

  lax.fori_loop(starts_ref[b], starts_ref[b + 1], body, 0)


def _tc_apply(starts, r8, cb, selid, tokv, lpv, tokens, logprobs):
  smem = pl.BlockSpec(memory_space=pltpu.SMEM)
  blk = pl.BlockSpec((_BLK_ROWS, MAX_TOKENS), lambda i: (i, 0))
  return pl.pallas_call(
      _tc_apply_body,
      grid=(_NBLK,),
      in_specs=[smem, smem, smem, smem, smem, smem, blk, blk],
      out_specs=[blk, blk],
      out_shape=(
          jax.ShapeDtypeStruct((MAX_SEQS, MAX_TOKENS), jnp.int32),
          jax.ShapeDtypeStruct((MAX_SEQS, MAX_TOKENS), jnp.float32),
      ),
      compiler_params=pltpu.CompilerParams(
          dimension_semantics=("arbitrary",),
      ),
  )(starts, r8, cb, selid, tokv, lpv, tokens, logprobs)


@jax.jit
def _kernel_impl(tokens, logprobs, num_tokens, local_seq_ids, new_tokens,
                 new_log_probs):
  cnt, starts, r8, cb, selid, tokv, lpv = _sc_plan(
      num_tokens, local_seq_ids, new_tokens, new_log_probs)
  tok_out, lp_out = _tc_apply(starts, r8, cb, selid, tokv, lpv, tokens,
                              logprobs)
  return tok_out, lp_out, cnt


def kernel(tokens, logprobs, num_tokens, local_seq_ids, new_tokens,
           new_log_probs, num_new_tokens):
  del num_new_tokens
  return _kernel_impl(tokens, logprobs, num_tokens, local_seq_ids, new_tokens,
                      new_log_probs)
